# baseline (speedup 1.0000x reference)
import functools

import jax
import jax.numpy as jnp
from jax.experimental import pallas as pl
from jax.experimental.pallas import tpu as pltpu

N_KEEP = 1000
N_PAD = 1024
ROW_TILE = 128
COL_CHUNK = 256


def _dense_kernel(boxes_r_ref, boxesT_ref, scores_r_ref, scoresT_ref,
                  classes_r_ref, classesT_ref,
                  W1_ref, b1_ref, W2_ref, b2_ref, W3_ref, b3_ref,
                  L1_ref, lb1_ref, L2_ref, lb2_ref,
                  out_ref):
    i = pl.program_id(0)

    boxes_r = boxes_r_ref[...]
    x1r = boxes_r[:, 0:1]
    y1r = boxes_r[:, 1:2]
    x2r = boxes_r[:, 2:3]
    y2r = boxes_r[:, 3:4]
    s_r = scores_r_ref[...]
    c_r = classes_r_ref[...]
    area_r = (x2r - x1r) * (y2r - y1r)

    row_ids = i * ROW_TILE + jax.lax.broadcasted_iota(
        jnp.int32, (ROW_TILE, 1), 0)

    W1 = W1_ref[...]
    b1 = b1_ref[...]
    W2 = W2_ref[...]
    b2 = b2_ref[...]
    W3 = W3_ref[...]
    b3 = b3_ref[...]

    S_acc = jnp.zeros((ROW_TILE, 1), jnp.float32)
    D_acc = jnp.zeros((ROW_TILE, 1), jnp.float32)

    for chunk in range(N_PAD // COL_CHUNK):
        c0 = chunk * COL_CHUNK
        x1c = boxesT_ref[0:1, c0:c0 + COL_CHUNK]
        y1c = boxesT_ref[1:2, c0:c0 + COL_CHUNK]
        x2c = boxesT_ref[2:3, c0:c0 + COL_CHUNK]
        y2c = boxesT_ref[3:4, c0:c0 + COL_CHUNK]
        s_c = scoresT_ref[0:1, c0:c0 + COL_CHUNK]
        c_c = classesT_ref[0:1, c0:c0 + COL_CHUNK]
        area_c = (x2c - x1c) * (y2c - y1c)

        w = jnp.maximum(jnp.minimum(x2r, x2c) - jnp.maximum(x1r, x1c), 0.0)
        h = jnp.maximum(jnp.minimum(y2r, y2c) - jnp.maximum(y1r, y1c), 0.0)
        inter = w * h
        union = area_r + area_c - inter
        iou = inter / (union + 1e-06)

        col_ids = c0 + jax.lax.broadcasted_iota(
            jnp.int32, (1, COL_CHUNK), 1)
        diag = row_ids == col_ids
        iou = jnp.where(diag, 0.0, iou)

        dx1 = jnp.abs(x1r - x1c)
        dy1 = jnp.abs(y1r - y1c)
        dx2 = jnp.abs(x2r - x2c)
        dy2 = jnp.abs(y2r - y2c)

        h2_acc = [None] * 16
        s_pre = None
        for k in range(32):
            rc = (b1[0, k] + W1[5, k] * s_r) + W1[6, k] * s_c
            h1k = jnp.maximum(
                W1[0, k] * iou + W1[1, k] * dx1 + W1[2, k] * dy1
                + W1[3, k] * dx2 + W1[4, k] * dy2 + rc, 0.0)
            for m in range(16):
                t = W2[k, m] * h1k
                h2_acc[m] = t if h2_acc[m] is None else h2_acc[m] + t
        for m in range(16):
            h2m = jnp.maximum(h2_acc[m] + b2[0, m], 0.0)
            t = W3[m, 0] * h2m
            s_pre = t if s_pre is None else s_pre + t
        s_ij = jax.nn.sigmoid(s_pre + b3[0, 0])

        mask = jnp.logical_and(c_r == c_c, s_c > s_r)
        contrib = jnp.where(mask, s_ij * iou, 0.0)
        S_acc = S_acc + jnp.sum(contrib, axis=1, keepdims=True)
        D_acc = D_acc + jnp.sum(iou, axis=1, keepdims=True)

    D = D_acc * (1.0 / N_KEEP)

    L1 = L1_ref[...]
    lb1 = lb1_ref[...]
    L2 = L2_ref[...]
    lb2 = lb2_ref[...]
    lam_cols = (x1r, y1r, x2r, y2r, s_r)
    lam_pre = None
    for t in range(16):
        acc = lb1[0, t]
        for c in range(5):
            acc = acc + L1[c, t] * lam_cols[c]
        ht = jnp.maximum(acc, 0.0)
        term = L2[t, 0] * ht
        lam_pre = term if lam_pre is None else lam_pre + term
    lam = jax.nn.sigmoid(lam_pre + lb2[0, 0])

    E = lam * S_acc * D
    new_s = s_r * jnp.exp(-E)
    out_ref[...] = jnp.where(row_ids < N_KEEP, new_s, -1.0)


@jax.jit
def _dense_stage(boxes_k, scores_k, classes_k,
                 W1, b1, W2, b2, W3, b3, L1, lb1, L2, lb2):
    pad = N_PAD - N_KEEP
    boxes_p = jnp.pad(boxes_k, ((0, pad), (0, 0)))
    scores_p = jnp.pad(scores_k, (0, pad), constant_values=-1.0)
    classes_p = jnp.pad(classes_k, (0, pad), constant_values=-1)

    boxesT = boxes_p.T
    scores_r = scores_p[:, None]
    scoresT = scores_p[None, :]
    classes_r = classes_p[:, None]
    classesT = classes_p[None, :]

    grid = (N_PAD // ROW_TILE,)
    row_spec2 = lambda w: pl.BlockSpec((ROW_TILE, w), lambda i: (i, 0))
    full = lambda a, b: pl.BlockSpec((a, b), lambda i: (0, 0))

    out = pl.pallas_call(
        _dense_kernel,
        grid=grid,
        in_specs=[
            row_spec2(4),
            full(4, N_PAD),
            row_spec2(1),
            full(1, N_PAD),
            row_spec2(1),
            full(1, N_PAD),
            full(7, 32), full(1, 32),
            full(32, 16), full(1, 16),
            full(16, 1), full(1, 1),
            full(5, 16), full(1, 16),
            full(16, 1), full(1, 1),
        ],
        out_specs=pl.BlockSpec((ROW_TILE, 1), lambda i: (i, 0)),
        out_shape=jax.ShapeDtypeStruct((N_PAD, 1), jnp.float32),
    )(boxes_p, boxesT, scores_r, scoresT, classes_r, classesT,
      W1, b1[None, :], W2, b2[None, :], W3, b3[None, :],
      L1, lb1[None, :], L2, lb2[None, :])
    return out[:N_KEEP, 0]


def kernel(boxes, scores, classes, W1, b1, W2, b2, W3, b3, L1, lb1, L2, lb2):
    scores_k, idx = jax.lax.top_k(scores, N_KEEP)
    boxes_k = boxes[idx]
    classes_k = classes[idx]
    new_scores = _dense_stage(boxes_k, scores_k, classes_k,
                              W1, b1, W2, b2, W3, b3, L1, lb1, L2, lb2)
    _, idx2 = jax.lax.top_k(new_scores, 50)
    return (boxes_k[idx2], new_scores[idx2], classes_k[idx2])
